# Initial kernel scaffold; baseline (speedup 1.0000x reference)
import jax
import jax.numpy as jnp
from jax.experimental import pallas as pl


def kernel(batch_input_feats, batch_input_labels, batch_input_labels_ori, batch_cent_feats, batch_cent_labels, batch_cent_labels_ori, W_self, b_self, W_neigh, b_neigh, bias, edge_index):
    raise NotImplementedError("write your pallas kernel here")


# speedup vs baseline: 5.4226x; 5.4226x over previous
import functools

import jax
import jax.numpy as jnp
from jax import lax
from jax.experimental import pallas as pl
from jax.experimental.pallas import tpu as pltpu
from jax.experimental.pallas import tpu_sc as plsc

D = 128
DP = 144
NC = 2
NS = 16
NW = NC * NS


def _sc_aggregate(x_pad, src, dst, zero_init):
    E = src.shape[0]
    NP = zero_init.shape[0] * NS
    CH = 80
    epw = E // NW
    n_chunks = epw // CH
    rpt = NP // NS

    mesh = plsc.VectorSubcoreMesh(
        core_axis_name="c", subcore_axis_name="s", num_cores=NC, num_subcores=NS
    )

    @functools.partial(
        pl.kernel,
        out_type=jax.ShapeDtypeStruct((NC, NP, DP), jnp.float32),
        mesh=mesh,
        scratch_types=[
            pltpu.VMEM_SHARED((NP, DP), jnp.float32),
            pltpu.VMEM((CH,), jnp.int32),
            pltpu.VMEM((CH,), jnp.int32),
            pltpu.VMEM((CH, DP), jnp.float32),
            pltpu.SemaphoreType.DMA,
        ],
        compiler_params=pltpu.CompilerParams(use_tc_tiling_on_sc=False),
    )
    def agg(x_hbm, src_hbm, dst_hbm, zero_hbm, out_hbm, acc, idx_s, idx_d, rows, sem):
        c = lax.axis_index("c")
        s = lax.axis_index("s")
        r0 = s * rpt
        pltpu.sync_copy(zero_hbm, acc.at[pl.ds(r0, rpt)])
        plsc.subcore_barrier()
        base = c * (E // NC) + s * epw

        def body(i, carry):
            off = base + i * CH
            pltpu.sync_copy(src_hbm.at[pl.ds(off, CH)], idx_s)
            pltpu.sync_copy(dst_hbm.at[pl.ds(off, CH)], idx_d)
            pltpu.async_copy(x_hbm.at[idx_s], rows, sem).wait()
            pltpu.sync_copy(rows, acc.at[idx_d], add=True)
            return carry

        lax.fori_loop(0, n_chunks, body, 0)
        plsc.subcore_barrier()
        pltpu.sync_copy(acc.at[pl.ds(r0, rpt)], out_hbm.at[c, pl.ds(r0, rpt)])

    return agg(x_pad, src, dst, zero_init)


def _tc_combine(p0, p1, x, W_self, b_self, W_neigh, b_neigh, bias):
    N = x.shape[0]
    BL = 1000
    grid = (N // BL,)

    def body(p0_ref, p1_ref, x_ref, ws_ref, bs_ref, wn_ref, bn_ref, b_ref, o_ref):
        p = p0_ref[...] + p1_ref[...]
        deg = jnp.sum(p[:, D:DP], axis=1, keepdims=True)
        hn = p[:, :D] / jnp.maximum(deg, 1.0)
        h_self = lax.dot_general(
            x_ref[...], ws_ref[...], (((1,), (1,)), ((), ())),
            preferred_element_type=jnp.float32,
        ) + bs_ref[...]
        h_neigh = lax.dot_general(
            hn, wn_ref[...], (((1,), (1,)), ((), ())),
            preferred_element_type=jnp.float32,
        ) + bn_ref[...]
        o_ref[...] = (h_self + h_neigh) * 0.5 + b_ref[...]

    blk = lambda shape: pl.BlockSpec(shape, lambda i: (0,) * len(shape))
    return pl.pallas_call(
        body,
        grid=grid,
        in_specs=[
            pl.BlockSpec((BL, DP), lambda i: (i, 0)),
            pl.BlockSpec((BL, DP), lambda i: (i, 0)),
            pl.BlockSpec((BL, D), lambda i: (i, 0)),
            blk((D, D)),
            blk((1, D)),
            blk((D, D)),
            blk((1, D)),
            blk((1, D)),
        ],
        out_specs=pl.BlockSpec((BL, D), lambda i: (i, 0)),
        out_shape=jax.ShapeDtypeStruct((N, D), jnp.float32),
    )(p0, p1, x, W_self, b_self.reshape(1, D), W_neigh, b_neigh.reshape(1, D),
      bias.reshape(1, D))


def kernel(batch_input_feats, batch_input_labels, batch_input_labels_ori,
           batch_cent_feats, batch_cent_labels, batch_cent_labels_ori,
           W_self, b_self, W_neigh, b_neigh, bias, edge_index):
    x = batch_input_feats
    N = x.shape[0]
    src = edge_index[0]
    dst = edge_index[1]
    pad = jnp.zeros((N, DP - D), x.dtype).at[:, 0].set(1.0)
    x_pad = jnp.concatenate([x, pad], axis=1)
    np_rows = ((N + 8 * NS - 1) // (8 * NS)) * 8 * NS
    zero_init = jnp.zeros((np_rows // NS, DP), jnp.float32)
    parts = _sc_aggregate(x_pad, src, dst, zero_init)
    return _tc_combine(parts[0], parts[1], x, W_self, b_self, W_neigh,
                       b_neigh, bias)


# speedup vs baseline: 6.2794x; 1.1580x over previous
import functools

import jax
import jax.numpy as jnp
from jax import lax
from jax.experimental import pallas as pl
from jax.experimental.pallas import tpu as pltpu
from jax.experimental.pallas import tpu_sc as plsc

D = 128
DP = 136
NC = 2
NS = 16
NW = NC * NS
CH = 80
NB = 2


def _sc_aggregate(x_pad, src_t, dst_t, zero_init):
    NP = zero_init.shape[0] * NS
    nch = src_t.shape[1]
    rpt = NP // NS

    mesh = plsc.VectorSubcoreMesh(
        core_axis_name="c", subcore_axis_name="s", num_cores=NC, num_subcores=NS
    )

    @functools.partial(
        pl.kernel,
        out_type=jax.ShapeDtypeStruct((NC, NP, DP), jnp.float32),
        mesh=mesh,
        scratch_types=[
            pltpu.VMEM_SHARED((NP, DP), jnp.float32),
            pltpu.VMEM((nch, CH), jnp.int32),
            pltpu.VMEM((nch, CH), jnp.int32),
            pltpu.VMEM((CH, DP), jnp.float32),
            pltpu.VMEM((CH, DP), jnp.float32),
            pltpu.SemaphoreType.DMA,
            pltpu.SemaphoreType.DMA,
            pltpu.SemaphoreType.DMA,
            pltpu.SemaphoreType.DMA,
        ],
        compiler_params=pltpu.CompilerParams(use_tc_tiling_on_sc=False),
    )
    def agg(x_hbm, src_hbm, dst_hbm, zero_hbm, out_hbm, acc, idx_s, idx_d,
            rows0, rows1, gsem0, gsem1, ssem0, ssem1):
        rows = (rows0, rows1)
        gsem = (gsem0, gsem1)
        ssem = (ssem0, ssem1)
        c = lax.axis_index("c")
        s = lax.axis_index("s")
        wid = c * NS + s
        r0 = s * rpt
        pltpu.sync_copy(zero_hbm, acc.at[pl.ds(r0, rpt)])
        pltpu.sync_copy(src_hbm.at[wid], idx_s)
        pltpu.sync_copy(dst_hbm.at[wid], idx_d)
        pltpu.async_copy(x_hbm.at[idx_s.at[0]], rows[0], gsem[0])
        plsc.subcore_barrier()

        def wait_gather(b, i):
            pltpu.make_async_copy(x_hbm.at[idx_s.at[i]], rows[b],
                                  gsem[b]).wait()

        def wait_scatter(b, i):
            pltpu.make_async_copy(rows[b], acc.at[idx_d.at[i]],
                                  ssem[b]).wait()

        wait_gather(0, 0)
        pltpu.async_copy(rows[0], acc.at[idx_d.at[0]], ssem[0], add=True)
        pltpu.async_copy(x_hbm.at[idx_s.at[1]], rows[1], gsem[1])

        def outer(k, carry):
            i = 2 * k + 1
            wait_gather(1, i)
            pltpu.async_copy(rows[1], acc.at[idx_d.at[i]], ssem[1], add=True)
            wait_scatter(0, i - 1)
            pltpu.async_copy(x_hbm.at[idx_s.at[i + 1]], rows[0], gsem[0])
            wait_gather(0, i + 1)
            pltpu.async_copy(rows[0], acc.at[idx_d.at[i + 1]], ssem[0],
                             add=True)
            wait_scatter(1, i)
            pltpu.async_copy(x_hbm.at[idx_s.at[i + 2]], rows[1], gsem[1])
            return carry

        lax.fori_loop(0, (nch - 2) // 2, outer, 0)

        wait_gather(1, nch - 1)
        pltpu.async_copy(rows[1], acc.at[idx_d.at[nch - 1]], ssem[1], add=True)
        wait_scatter(0, nch - 2)
        wait_scatter(1, nch - 1)
        plsc.subcore_barrier()
        pltpu.sync_copy(acc.at[pl.ds(r0, rpt)], out_hbm.at[c, pl.ds(r0, rpt)])

    return agg(x_pad, src_t, dst_t, zero_init)


def _tc_combine(p0, p1, x, W_self, b_self, W_neigh, b_neigh, bias):
    N = x.shape[0]
    BL = 1000
    grid = (N // BL,)

    def body(p0_ref, p1_ref, x_ref, ws_ref, bs_ref, wn_ref, bn_ref, b_ref, o_ref):
        p = p0_ref[...] + p1_ref[...]
        deg = jnp.sum(p[:, D:DP], axis=1, keepdims=True)
        hn = p[:, :D] / jnp.maximum(deg, 1.0)
        h_self = lax.dot_general(
            x_ref[...], ws_ref[...], (((1,), (1,)), ((), ())),
            preferred_element_type=jnp.float32,
        ) + bs_ref[...]
        h_neigh = lax.dot_general(
            hn, wn_ref[...], (((1,), (1,)), ((), ())),
            preferred_element_type=jnp.float32,
        ) + bn_ref[...]
        o_ref[...] = (h_self + h_neigh) * 0.5 + b_ref[...]

    blk = lambda shape: pl.BlockSpec(shape, lambda i: (0,) * len(shape))
    return pl.pallas_call(
        body,
        grid=grid,
        in_specs=[
            pl.BlockSpec((BL, DP), lambda i: (i, 0)),
            pl.BlockSpec((BL, DP), lambda i: (i, 0)),
            pl.BlockSpec((BL, D), lambda i: (i, 0)),
            blk((D, D)),
            blk((1, D)),
            blk((D, D)),
            blk((1, D)),
            blk((1, D)),
        ],
        out_specs=pl.BlockSpec((BL, D), lambda i: (i, 0)),
        out_shape=jax.ShapeDtypeStruct((N, D), jnp.float32),
    )(p0, p1, x, W_self, b_self.reshape(1, D), W_neigh, b_neigh.reshape(1, D),
      bias.reshape(1, D))


def kernel(batch_input_feats, batch_input_labels, batch_input_labels_ori,
           batch_cent_feats, batch_cent_labels, batch_cent_labels_ori,
           W_self, b_self, W_neigh, b_neigh, bias, edge_index):
    x = batch_input_feats
    N = x.shape[0]
    E = edge_index.shape[1]
    src = edge_index[0]
    dst = edge_index[1]
    pad = jnp.zeros((N, DP - D), x.dtype).at[:, 0].set(1.0)
    x_pad = jnp.concatenate([x, pad], axis=1)
    np_rows = ((N + 1 + 8 * NS - 1) // (8 * NS)) * 8 * NS
    zero_init = jnp.zeros((np_rows // NS, DP), jnp.float32)
    epw = E // NW
    epw_p = ((epw + 2 * CH - 1) // (2 * CH)) * 2 * CH
    src_t = jnp.pad(src.reshape(NW, epw), ((0, 0), (0, epw_p - epw)))
    dst_t = jnp.pad(dst.reshape(NW, epw), ((0, 0), (0, epw_p - epw)),
                    constant_values=np_rows - 1)
    src_t = src_t.reshape(NW, epw_p // CH, CH)
    dst_t = dst_t.reshape(NW, epw_p // CH, CH)
    parts = _sc_aggregate(x_pad, src_t, dst_t, zero_init)
    return _tc_combine(parts[0], parts[1], x, W_self, b_self, W_neigh,
                       b_neigh, bias)


# speedup vs baseline: 11.2105x; 1.7853x over previous
import functools

import jax
import jax.numpy as jnp
from jax import lax
from jax.experimental import pallas as pl
from jax.experimental.pallas import tpu as pltpu
from jax.experimental.pallas import tpu_sc as plsc

D = 128
DG = 8
NC = 2
NS = 16
NW = NC * NS
CH = 80


def _sc_aggregate(x, e4, ones_rows, zf, zd):
    N = x.shape[0]
    nch = e4.shape[2]
    NP = zf.shape[0] * NS
    rpt = NP // NS

    mesh = plsc.VectorSubcoreMesh(
        core_axis_name="c", subcore_axis_name="s", num_cores=NC, num_subcores=NS
    )

    @functools.partial(
        pl.kernel,
        out_type=(
            jax.ShapeDtypeStruct((NC, NP, D), jnp.float32),
            jax.ShapeDtypeStruct((NC, NP, DG), jnp.float32),
        ),
        mesh=mesh,
        scratch_types=[
            pltpu.VMEM_SHARED((NP, D), jnp.float32),
            pltpu.VMEM_SHARED((NP, DG), jnp.float32),
            pltpu.VMEM((nch, CH), jnp.int32),
            pltpu.VMEM((nch, CH), jnp.int32),
            pltpu.VMEM((CH, D), jnp.float32),
            pltpu.VMEM((CH, D), jnp.float32),
            pltpu.VMEM((CH, DG), jnp.float32),
            pltpu.SemaphoreType.DMA,
            pltpu.SemaphoreType.DMA,
            pltpu.SemaphoreType.DMA,
            pltpu.SemaphoreType.DMA,
            pltpu.SemaphoreType.DMA,
            pltpu.SemaphoreType.DMA,
        ],
        compiler_params=pltpu.CompilerParams(use_tc_tiling_on_sc=False),
    )
    def agg(x_hbm, e_hbm, ones_hbm, zf_hbm, zd_hbm, of_hbm, od_hbm,
            facc, dacc, idx_s, idx_d, rows0, rows1, ones_v,
            gsem0, gsem1, fsem0, fsem1, dsem0, dsem1):
        rows = (rows0, rows1)
        gsem = (gsem0, gsem1)
        fsem = (fsem0, fsem1)
        dsem = (dsem0, dsem1)
        c = lax.axis_index("c")
        s = lax.axis_index("s")
        wid = c * NS + s
        r0 = s * rpt
        pltpu.sync_copy(zf_hbm, facc.at[pl.ds(r0, rpt)])
        pltpu.sync_copy(zd_hbm, dacc.at[pl.ds(r0, rpt)])
        pltpu.sync_copy(e_hbm.at[0, wid], idx_s)
        pltpu.sync_copy(e_hbm.at[1, wid], idx_d)
        pltpu.sync_copy(ones_hbm, ones_v)
        pltpu.async_copy(x_hbm.at[idx_s.at[0]], rows[0], gsem[0])
        plsc.subcore_barrier()

        def wait_gather(b, i):
            pltpu.make_async_copy(x_hbm.at[idx_s.at[i]], rows[b],
                                  gsem[b]).wait()

        def start_scatter(b, i):
            pltpu.async_copy(rows[b], facc.at[idx_d.at[i]], fsem[b], add=True)
            pltpu.async_copy(ones_v, dacc.at[idx_d.at[i]], dsem[b], add=True)

        def wait_scatter(b, i):
            pltpu.make_async_copy(rows[b], facc.at[idx_d.at[i]],
                                  fsem[b]).wait()
            pltpu.make_async_copy(ones_v, dacc.at[idx_d.at[i]],
                                  dsem[b]).wait()

        wait_gather(0, 0)
        start_scatter(0, 0)
        pltpu.async_copy(x_hbm.at[idx_s.at[1]], rows[1], gsem[1])

        def outer(k, carry):
            i = 2 * k + 1
            wait_gather(1, i)
            start_scatter(1, i)
            wait_scatter(0, i - 1)
            pltpu.async_copy(x_hbm.at[idx_s.at[i + 1]], rows[0], gsem[0])
            wait_gather(0, i + 1)
            start_scatter(0, i + 1)
            wait_scatter(1, i)

            @pl.when(i + 2 < nch)
            def _():
                pltpu.async_copy(x_hbm.at[idx_s.at[i + 2]], rows[1], gsem[1])

            return carry

        lax.fori_loop(0, (nch - 1) // 2, outer, 0)
        wait_scatter(0, nch - 1)
        plsc.subcore_barrier()
        pltpu.sync_copy(facc.at[pl.ds(r0, rpt)], of_hbm.at[c, pl.ds(r0, rpt)])
        pltpu.sync_copy(dacc.at[pl.ds(r0, rpt)], od_hbm.at[c, pl.ds(r0, rpt)])

    return agg(x, e4, ones_rows, zf, zd)


def _tc_combine(feats, degs, x, W_self, b_self, W_neigh, b_neigh, bias):
    N = x.shape[0]
    BL = 1000
    grid = (N // BL,)

    def body(f_ref, d_ref, x_ref, ws_ref, bs_ref, wn_ref, bn_ref, b_ref, o_ref):
        p = f_ref[0] + f_ref[1]
        d = d_ref[0] + d_ref[1]
        deg = jnp.sum(d, axis=1, keepdims=True)
        hn = p / jnp.maximum(deg, 1.0)
        h_self = lax.dot_general(
            x_ref[...], ws_ref[...], (((1,), (1,)), ((), ())),
            preferred_element_type=jnp.float32,
        ) + bs_ref[...]
        h_neigh = lax.dot_general(
            hn, wn_ref[...], (((1,), (1,)), ((), ())),
            preferred_element_type=jnp.float32,
        ) + bn_ref[...]
        o_ref[...] = (h_self + h_neigh) * 0.5 + b_ref[...]

    blk = lambda shape: pl.BlockSpec(shape, lambda i: (0,) * len(shape))
    return pl.pallas_call(
        body,
        grid=grid,
        in_specs=[
            pl.BlockSpec((NC, BL, D), lambda i: (0, i, 0)),
            pl.BlockSpec((NC, BL, DG), lambda i: (0, i, 0)),
            pl.BlockSpec((BL, D), lambda i: (i, 0)),
            blk((D, D)),
            blk((1, D)),
            blk((D, D)),
            blk((1, D)),
            blk((1, D)),
        ],
        out_specs=pl.BlockSpec((BL, D), lambda i: (i, 0)),
        out_shape=jax.ShapeDtypeStruct((N, D), jnp.float32),
    )(feats, degs, x, W_self, b_self.reshape(1, D), W_neigh,
      b_neigh.reshape(1, D), bias.reshape(1, D))


def kernel(batch_input_feats, batch_input_labels, batch_input_labels_ori,
           batch_cent_feats, batch_cent_labels, batch_cent_labels_ori,
           W_self, b_self, W_neigh, b_neigh, bias, edge_index):
    x = batch_input_feats
    N = x.shape[0]
    E = edge_index.shape[1]
    epw = E // NW
    e4 = edge_index.reshape(2, NW, epw // CH, CH)
    np_rows = ((N + 8 * NS - 1) // (8 * NS)) * 8 * NS
    zf = jnp.zeros((np_rows // NS, D), jnp.float32)
    zd = jnp.zeros((np_rows // NS, DG), jnp.float32)
    ones_rows = jnp.full((CH, DG), 1.0 / DG, jnp.float32)
    feats, degs = _sc_aggregate(x, e4, ones_rows, zf, zd)
    return _tc_combine(feats, degs, x, W_self, b_self, W_neigh, b_neigh, bias)


# speedup vs baseline: 15.3993x; 1.3737x over previous
import functools

import jax
import jax.numpy as jnp
from jax import lax
from jax.experimental import pallas as pl
from jax.experimental.pallas import tpu as pltpu
from jax.experimental.pallas import tpu_sc as plsc

D = 128
DG = 8
NC = 2
NS = 16
NW = NC * NS
CH = 80
NB = 3


def _sc_aggregate(x, e4, ones_rows, zf, zd):
    nch = e4.shape[2]
    NP = zf.shape[0] * NS
    rpt = NP // NS

    mesh = plsc.VectorSubcoreMesh(
        core_axis_name="c", subcore_axis_name="s", num_cores=NC, num_subcores=NS
    )

    @functools.partial(
        pl.kernel,
        out_type=(
            jax.ShapeDtypeStruct((NC, NP, D), jnp.float32),
            jax.ShapeDtypeStruct((NC, NP, DG), jnp.float32),
        ),
        mesh=mesh,
        scratch_types=[
            pltpu.VMEM_SHARED((NP, D), jnp.float32),
            pltpu.VMEM_SHARED((NP, DG), jnp.float32),
            pltpu.VMEM((nch, CH), jnp.int32),
            pltpu.VMEM((CH, DG), jnp.float32),
        ]
        + [pltpu.VMEM((CH,), jnp.int32) for _ in range(NB)]
        + [pltpu.VMEM((CH, D), jnp.float32) for _ in range(NB)]
        + [pltpu.SemaphoreType.DMA for _ in range(4 * NB)],
        compiler_params=pltpu.CompilerParams(use_tc_tiling_on_sc=False),
    )
    def agg(x_hbm, e_hbm, ones_hbm, zf_hbm, zd_hbm, of_hbm, od_hbm,
            facc, dacc, idx_d, ones_v, *rest):
        sbuf = rest[:NB]
        rows = rest[NB:2 * NB]
        isem = rest[2 * NB:3 * NB]
        gsem = rest[3 * NB:4 * NB]
        fsem = rest[4 * NB:5 * NB]
        dsem = rest[5 * NB:6 * NB]
        c = lax.axis_index("c")
        s = lax.axis_index("s")
        wid = c * NS + s
        r0 = s * rpt
        pltpu.sync_copy(zf_hbm, facc.at[pl.ds(r0, rpt)])
        pltpu.sync_copy(zd_hbm, dacc.at[pl.ds(r0, rpt)])
        pltpu.sync_copy(e_hbm.at[1, wid], idx_d)
        pltpu.sync_copy(ones_hbm, ones_v)
        for b in range(NB):
            pltpu.sync_copy(e_hbm.at[0, wid, b], sbuf[b])
        pltpu.async_copy(x_hbm.at[sbuf[0]], rows[0], gsem[0])
        pltpu.async_copy(x_hbm.at[sbuf[1]], rows[1], gsem[1])
        pltpu.async_copy(x_hbm.at[sbuf[2]], rows[2], gsem[2])
        plsc.subcore_barrier()

        def wait_gather(b, i):
            pltpu.make_async_copy(x_hbm.at[sbuf[b]], rows[b], gsem[b]).wait()

        def start_scatter(b, i):
            pltpu.async_copy(rows[b], facc.at[idx_d.at[i]], fsem[b], add=True)
            pltpu.async_copy(ones_v, dacc.at[idx_d.at[i]], dsem[b], add=True)

        def wait_scatter(b, i):
            pltpu.make_async_copy(rows[b], facc.at[idx_d.at[i]],
                                  fsem[b]).wait()
            pltpu.make_async_copy(ones_v, dacc.at[idx_d.at[i]],
                                  dsem[b]).wait()

        def start_fetch(b, i):
            pltpu.async_copy(e_hbm.at[0, wid, i], sbuf[b], isem[b])

        def wait_fetch(b, i):
            pltpu.make_async_copy(e_hbm.at[0, wid, i], sbuf[b],
                                  isem[b]).wait()

        def sub(i, b):
            wait_gather(b, i)
            start_scatter(b, i)

            @pl.when(i + NB < nch)
            def _():
                start_fetch(b, i + NB)

            @pl.when(i + 2 < nch)
            def _():
                b2 = (b + 2) % NB
                wait_fetch(b2, i + 2)
                wait_scatter((b - 1) % NB, i - 1)
                pltpu.async_copy(x_hbm.at[sbuf[b2]], rows[b2], gsem[b2])

        wait_gather(0, 0)
        start_scatter(0, 0)
        start_fetch(0, NB)

        def outer(k, carry):
            i = NB * k + 1
            sub(i, 1)
            sub(i + 1, 2)
            sub(i + 2, 0)
            return carry

        lax.fori_loop(0, (nch - 2) // NB, outer, 0)
        bl = (nch - 1) % NB
        wait_gather(bl, nch - 1)
        start_scatter(bl, nch - 1)
        for i in (nch - 3, nch - 2, nch - 1):
            wait_scatter(i % NB, i)
        plsc.subcore_barrier()
        pltpu.sync_copy(facc.at[pl.ds(r0, rpt)], of_hbm.at[c, pl.ds(r0, rpt)])
        pltpu.sync_copy(dacc.at[pl.ds(r0, rpt)], od_hbm.at[c, pl.ds(r0, rpt)])

    return agg(x, e4, ones_rows, zf, zd)


def _tc_combine(feats, degs, x, W_self, b_self, W_neigh, b_neigh, bias):
    N = x.shape[0]
    BL = 1000
    grid = (N // BL,)

    def body(f_ref, d_ref, x_ref, ws_ref, bs_ref, wn_ref, bn_ref, b_ref, o_ref):
        p = f_ref[0] + f_ref[1]
        d = d_ref[0] + d_ref[1]
        deg = jnp.sum(d, axis=1, keepdims=True)
        hn = p / jnp.maximum(deg, 1.0)
        h_self = lax.dot_general(
            x_ref[...], ws_ref[...], (((1,), (1,)), ((), ())),
            preferred_element_type=jnp.float32,
        ) + bs_ref[...]
        h_neigh = lax.dot_general(
            hn, wn_ref[...], (((1,), (1,)), ((), ())),
            preferred_element_type=jnp.float32,
        ) + bn_ref[...]
        o_ref[...] = (h_self + h_neigh) * 0.5 + b_ref[...]

    blk = lambda shape: pl.BlockSpec(shape, lambda i: (0,) * len(shape))
    return pl.pallas_call(
        body,
        grid=grid,
        in_specs=[
            pl.BlockSpec((NC, BL, D), lambda i: (0, i, 0)),
            pl.BlockSpec((NC, BL, DG), lambda i: (0, i, 0)),
            pl.BlockSpec((BL, D), lambda i: (i, 0)),
            blk((D, D)),
            blk((1, D)),
            blk((D, D)),
            blk((1, D)),
            blk((1, D)),
        ],
        out_specs=pl.BlockSpec((BL, D), lambda i: (i, 0)),
        out_shape=jax.ShapeDtypeStruct((N, D), jnp.float32),
    )(feats, degs, x, W_self, b_self.reshape(1, D), W_neigh,
      b_neigh.reshape(1, D), bias.reshape(1, D))


def kernel(batch_input_feats, batch_input_labels, batch_input_labels_ori,
           batch_cent_feats, batch_cent_labels, batch_cent_labels_ori,
           W_self, b_self, W_neigh, b_neigh, bias, edge_index):
    x = batch_input_feats
    N = x.shape[0]
    E = edge_index.shape[1]
    epw = E // NW
    e4 = edge_index.reshape(2, NW, epw // CH, CH)
    np_rows = ((N + 8 * NS - 1) // (8 * NS)) * 8 * NS
    zf = jnp.zeros((np_rows // NS, D), jnp.float32)
    zd = jnp.zeros((np_rows // NS, DG), jnp.float32)
    ones_rows = jnp.full((CH, DG), 1.0 / DG, jnp.float32)
    feats, degs = _sc_aggregate(x, e4, ones_rows, zf, zd)
    return _tc_combine(feats, degs, x, W_self, b_self, W_neigh, b_neigh, bias)
